# baseline (speedup 1.0000x reference)
import functools

import jax
import jax.numpy as jnp
from jax import lax
from jax.experimental import pallas as pl
from jax.experimental.pallas import tpu as pltpu
from jax.experimental.pallas import tpu_sc as plsc

BATCH = 16384
HALF = BATCH // 2
EMBED = 64
NC = 2
NS = 16
LANES = 16
NW = NC * NS
P_PER_W = HALF // NW
CHUNK = 64
N_CHUNKS = P_PER_W // CHUNK
GROUPS = CHUNK // LANES

TC_STEP = 64


def _sc_gather_body(users_hbm, ut_hbm, out_hbm, idxa_v, idxb_v, buf_v,
                    out_sem, sem):
    wid = lax.axis_index("s") * NC + lax.axis_index("c")
    base = wid * P_PER_W
    lane = lax.iota(jnp.int32, LANES)

    def chunk_body(ch, _):
        cbase = base + ch * CHUNK
        pltpu.sync_copy(users_hbm.at[pl.ds(cbase, CHUNK)], idxa_v)
        pltpu.sync_copy(users_hbm.at[pl.ds(cbase + HALF, CHUNK)], idxb_v)
        copies = []
        for g in range(GROUPS):
            avec = idxa_v[pl.ds(g * LANES, LANES)]
            bvec = idxb_v[pl.ds(g * LANES, LANES)]
            for j in range(LANES):
                ra = jnp.sum(jnp.where(lane == j, avec, 0))
                rb = jnp.sum(jnp.where(lane == j, bvec, 0))
                r = g * LANES + j
                copies.append(pltpu.async_copy(
                    ut_hbm.at[ra], buf_v.at[r, pl.ds(0, EMBED)], sem))
                copies.append(pltpu.async_copy(
                    ut_hbm.at[rb], buf_v.at[r, pl.ds(EMBED, EMBED)], sem))
        for c in copies:
            c.wait()
        pltpu.async_copy(
            buf_v, out_hbm.at[pl.ds(cbase, CHUNK)], out_sem).wait()
        return 0

    lax.fori_loop(0, N_CHUNKS, chunk_body, 0)


def _sc_gather(users, user_table):
    mesh = plsc.VectorSubcoreMesh(core_axis_name="c", subcore_axis_name="s")
    k = pl.kernel(
        _sc_gather_body,
        out_type=jax.ShapeDtypeStruct((HALF, 2 * EMBED), jnp.float32),
        mesh=mesh,
        compiler_params=pltpu.CompilerParams(needs_layout_passes=False),
        scratch_types=[
            pltpu.VMEM((CHUNK,), jnp.int32),
            pltpu.VMEM((CHUNK,), jnp.int32),
            pltpu.VMEM((CHUNK, 2 * EMBED), jnp.float32),
            pltpu.SemaphoreType.DMA,
            pltpu.SemaphoreType.DMA,
        ],
    )
    return k(users, user_table)


def _tc_gather_body(idx_ref, it_hbm, cola_ref, colb_ref, out_ref,
                    bufa_v, bufb_v, sem):
    step = pl.program_id(0)
    copies = []
    for j in range(TC_STEP):
        e = step * TC_STEP + j
        ra = idx_ref[e] >> 1
        rb = idx_ref[e + HALF] >> 1
        copies.append(pltpu.make_async_copy(it_hbm.at[ra], bufa_v.at[j], sem))
        copies.append(pltpu.make_async_copy(it_hbm.at[rb], bufb_v.at[j], sem))
    for c in copies:
        c.start()
    for c in copies:
        c.wait()
    para = (cola_ref[...] & 1) == 1
    parb = (colb_ref[...] & 1) == 1
    a = jnp.where(para, bufa_v[:, EMBED:], bufa_v[:, :EMBED])
    b = jnp.where(parb, bufb_v[:, EMBED:], bufb_v[:, :EMBED])
    out_ref[...] = jnp.concatenate([a, b], axis=1)


def _tc_gather(items, item_table):
    it2 = item_table.reshape(-1, 2 * EMBED)
    cola = items[:HALF].reshape(-1, 1)
    colb = items[HALF:].reshape(-1, 1)
    grid_spec = pltpu.PrefetchScalarGridSpec(
        num_scalar_prefetch=1,
        grid=(HALF // TC_STEP,),
        in_specs=[
            pl.BlockSpec(memory_space=pl.ANY),
            pl.BlockSpec((TC_STEP, 1), lambda i, idx: (i, 0)),
            pl.BlockSpec((TC_STEP, 1), lambda i, idx: (i, 0)),
        ],
        out_specs=pl.BlockSpec((TC_STEP, 2 * EMBED), lambda i, idx: (i, 0)),
        scratch_shapes=[
            pltpu.VMEM((TC_STEP, 2 * EMBED), jnp.float32),
            pltpu.VMEM((TC_STEP, 2 * EMBED), jnp.float32),
            pltpu.SemaphoreType.DMA,
        ],
    )
    return pl.pallas_call(
        _tc_gather_body,
        grid_spec=grid_spec,
        out_shape=jax.ShapeDtypeStruct((HALF, 2 * EMBED), jnp.float32),
    )(items, it2, cola, colb)


def _tc_score_body(u_ref, i_ref, o_ref):
    prod = u_ref[...] * i_ref[...]
    s0 = jnp.sum(prod[:, 0:EMBED], axis=1, keepdims=True)
    s1 = jnp.sum(prod[:, EMBED:2 * EMBED], axis=1, keepdims=True)
    s = jnp.concatenate([s0, s1], axis=1)
    o_ref[...] = 1.0 / (1.0 + jnp.exp(-s))


def _tc_score(u_s, i_s):
    blk = 1024
    return pl.pallas_call(
        _tc_score_body,
        grid=(HALF // blk,),
        in_specs=[
            pl.BlockSpec((blk, 2 * EMBED), lambda i: (i, 0)),
            pl.BlockSpec((blk, 2 * EMBED), lambda i: (i, 0)),
        ],
        out_specs=pl.BlockSpec((blk, 2), lambda i: (i, 0)),
        out_shape=jax.ShapeDtypeStruct((HALF, 2), jnp.float32),
    )(u_s, i_s)


@jax.jit
def kernel(users, items, user_table, item_table):
    u_s = _sc_gather(users, user_table)
    i_s = _tc_gather(items, item_table)
    scores2 = _tc_score(u_s, i_s)
    return jnp.concatenate([scores2[:, 0], scores2[:, 1]])


# speedup vs baseline: 1.4798x; 1.4798x over previous
import jax
import jax.numpy as jnp
from jax import lax
from jax.experimental import pallas as pl
from jax.experimental.pallas import tpu as pltpu
from jax.experimental.pallas import tpu_sc as plsc

BATCH = 16384
EMBED = 64
NC = 2
NS = 16
NW = NC * NS
E_PER_W = BATCH // NW
CHUNK = 128
N_CHUNKS = E_PER_W // CHUNK


def _sc_gather_body(users_hbm, items_hbm, ut_hbm, it_hbm, outu_hbm, outi_hbm,
                    idxu_v, idxi_v, bufu_v, bufi_v, gsem, osem):
    wid = lax.axis_index("s") * NC + lax.axis_index("c")
    base = wid * E_PER_W

    def chunk_body(ch, _):
        cbase = base + ch * CHUNK
        pltpu.sync_copy(users_hbm.at[pl.ds(cbase, CHUNK)], idxu_v)
        pltpu.sync_copy(items_hbm.at[pl.ds(cbase, CHUNK)], idxi_v)
        copies = []
        for g in range(CHUNK // 16):
            uvec = idxu_v[pl.ds(g * 16, 16)]
            ivec = idxi_v[pl.ds(g * 16, 16)]
            for j in range(16):
                r = g * 16 + j
                ru = uvec[j]
                ri = ivec[j]
                copies.append(
                    pltpu.async_copy(ut_hbm.at[ru], bufu_v.at[r], gsem))
                copies.append(
                    pltpu.async_copy(it_hbm.at[ri], bufi_v.at[r], gsem))
        for c in copies:
            c.wait()
        ou = pltpu.async_copy(bufu_v, outu_hbm.at[pl.ds(cbase, CHUNK)], osem)
        oi = pltpu.async_copy(bufi_v, outi_hbm.at[pl.ds(cbase, CHUNK)], osem)
        ou.wait()
        oi.wait()
        return 0

    lax.fori_loop(0, N_CHUNKS, chunk_body, 0)


def _sc_gather(users, items, user_table, item_table):
    mesh = plsc.VectorSubcoreMesh(core_axis_name="c", subcore_axis_name="s")
    k = pl.kernel(
        _sc_gather_body,
        out_type=[
            jax.ShapeDtypeStruct((BATCH, EMBED), jnp.float32),
            jax.ShapeDtypeStruct((BATCH, EMBED), jnp.float32),
        ],
        mesh=mesh,
        compiler_params=pltpu.CompilerParams(use_tc_tiling_on_sc=True),
        scratch_types=[
            pltpu.VMEM((CHUNK,), jnp.int32),
            pltpu.VMEM((CHUNK,), jnp.int32),
            pltpu.VMEM((CHUNK, EMBED), jnp.float32),
            pltpu.VMEM((CHUNK, EMBED), jnp.float32),
            pltpu.SemaphoreType.DMA,
            pltpu.SemaphoreType.DMA,
        ],
    )
    return k(users, items, user_table, item_table)


def _tc_score_body(u_ref, i_ref, o_ref):
    prod = u_ref[...] * i_ref[...]
    s = jnp.sum(prod, axis=1, keepdims=True)
    o_ref[...] = 1.0 / (1.0 + jnp.exp(-s))


def _tc_score(u_s, i_s):
    blk = 2048
    return pl.pallas_call(
        _tc_score_body,
        grid=(BATCH // blk,),
        in_specs=[
            pl.BlockSpec((blk, EMBED), lambda i: (i, 0)),
            pl.BlockSpec((blk, EMBED), lambda i: (i, 0)),
        ],
        out_specs=pl.BlockSpec((blk, 1), lambda i: (i, 0)),
        out_shape=jax.ShapeDtypeStruct((BATCH, 1), jnp.float32),
    )(u_s, i_s)


@jax.jit
def kernel(users, items, user_table, item_table):
    u_s, i_s = _sc_gather(users, items, user_table, item_table)
    return _tc_score(u_s, i_s).reshape(BATCH)


# speedup vs baseline: 2.0525x; 1.3870x over previous
import jax
import jax.numpy as jnp
from jax import lax
from jax.experimental import pallas as pl
from jax.experimental.pallas import tpu as pltpu
from jax.experimental.pallas import tpu_sc as plsc

BATCH = 16384
EMBED = 64
LANES = 16
NC = 2
NS = 16
NW = NC * NS
E_PER_W = BATCH // NW
CHUNK = 64
N_CHUNKS = E_PER_W // CHUNK
G = 4


def _extract_col(slab_v, j, col, out_v, e):
    for g4 in range(EMBED // LANES):
        rows = lax.iota(jnp.int32, LANES) + g4 * LANES
        sel = plsc.load_gather(
            slab_v,
            [jnp.full((LANES,), j, jnp.int32),
             rows,
             jnp.full((LANES,), col, jnp.int32)])
        out_v[e, pl.ds(g4 * LANES, LANES)] = sel


def _sc_gather_body(users_hbm, items_hbm, ut_hbm, it_hbm, outu_hbm, outi_hbm,
                    idxu_v, idxi_v, su_v, si_v, obu_v, obi_v, gsem, osem):
    wid = lax.axis_index("s") * NC + lax.axis_index("c")
    base = wid * E_PER_W

    def chunk_body(ch, _):
        cbase = base + ch * CHUNK
        pltpu.sync_copy(users_hbm.at[pl.ds(cbase, CHUNK)], idxu_v)
        pltpu.sync_copy(items_hbm.at[pl.ds(cbase, CHUNK)], idxi_v)
        for g16 in range(CHUNK // LANES):
            uvec = idxu_v[pl.ds(g16 * LANES, LANES)]
            ivec = idxi_v[pl.ds(g16 * LANES, LANES)]
            for q in range(LANES // G):
                copies = []
                for j in range(G):
                    e = q * G + j
                    au = pl.multiple_of((uvec[e] >> 7) * 128, 128)
                    ai = pl.multiple_of((ivec[e] >> 7) * 128, 128)
                    copies.append(pltpu.async_copy(
                        ut_hbm.at[:, pl.ds(au, 128)], su_v.at[j], gsem))
                    copies.append(pltpu.async_copy(
                        it_hbm.at[:, pl.ds(ai, 128)], si_v.at[j], gsem))
                for c in copies:
                    c.wait()
                for j in range(G):
                    e = q * G + j
                    _extract_col(su_v, j, uvec[e] & 127, obu_v,
                                 g16 * LANES + e)
                    _extract_col(si_v, j, ivec[e] & 127, obi_v,
                                 g16 * LANES + e)
        ou = pltpu.async_copy(obu_v, outu_hbm.at[pl.ds(cbase, CHUNK)], osem)
        oi = pltpu.async_copy(obi_v, outi_hbm.at[pl.ds(cbase, CHUNK)], osem)
        ou.wait()
        oi.wait()
        return 0

    lax.fori_loop(0, N_CHUNKS, chunk_body, 0)


def _sc_gather(users, items, user_table_t, item_table_t):
    mesh = plsc.VectorSubcoreMesh(core_axis_name="c", subcore_axis_name="s")
    k = pl.kernel(
        _sc_gather_body,
        out_type=[
            jax.ShapeDtypeStruct((BATCH, EMBED), jnp.float32),
            jax.ShapeDtypeStruct((BATCH, EMBED), jnp.float32),
        ],
        mesh=mesh,
        compiler_params=pltpu.CompilerParams(
            use_tc_tiling_on_sc=True, needs_layout_passes=False),
        scratch_types=[
            pltpu.VMEM((CHUNK,), jnp.int32),
            pltpu.VMEM((CHUNK,), jnp.int32),
            pltpu.VMEM((G, EMBED, 128), jnp.float32),
            pltpu.VMEM((G, EMBED, 128), jnp.float32),
            pltpu.VMEM((CHUNK, EMBED), jnp.float32),
            pltpu.VMEM((CHUNK, EMBED), jnp.float32),
            pltpu.SemaphoreType.DMA,
            pltpu.SemaphoreType.DMA,
        ],
    )
    return k(users, items, user_table_t, item_table_t)


def _tc_score_body(u_ref, i_ref, o_ref):
    prod = u_ref[...] * i_ref[...]
    s = jnp.sum(prod, axis=1, keepdims=True)
    o_ref[...] = 1.0 / (1.0 + jnp.exp(-s))


def _tc_score(u_s, i_s):
    blk = 2048
    return pl.pallas_call(
        _tc_score_body,
        grid=(BATCH // blk,),
        in_specs=[
            pl.BlockSpec((blk, EMBED), lambda i: (i, 0)),
            pl.BlockSpec((blk, EMBED), lambda i: (i, 0)),
        ],
        out_specs=pl.BlockSpec((blk, 1), lambda i: (i, 0)),
        out_shape=jax.ShapeDtypeStruct((BATCH, 1), jnp.float32),
    )(u_s, i_s)


@jax.jit
def kernel(users, items, user_table, item_table):
    u_s, i_s = _sc_gather(users, items, user_table.T, item_table.T)
    return _tc_score(u_s, i_s).reshape(BATCH)


# speedup vs baseline: 2.4203x; 1.1792x over previous
import jax
import jax.numpy as jnp
from jax import lax
from jax.experimental import pallas as pl
from jax.experimental.pallas import tpu as pltpu
from jax.experimental.pallas import tpu_sc as plsc

BATCH = 16384
EMBED = 64
LANES = 16
NC = 2
NS = 16
NW = NC * NS
E_PER_W = BATCH // NW
CHUNK = 64
N_CHUNKS = E_PER_W // CHUNK
DEPTH = 3
PAIR = 2


def _extract_col(slab_v, j, col, out_v, e):
    for g4 in range(EMBED // LANES):
        rows = lax.iota(jnp.int32, LANES) + g4 * LANES
        sel = plsc.load_gather(
            slab_v,
            [jnp.full((LANES,), j, jnp.int32),
             rows,
             jnp.full((LANES,), col, jnp.int32)])
        out_v[e, pl.ds(g4 * LANES, LANES)] = sel


def _sc_gather_body(users_hbm, items_hbm, ut_hbm, it_hbm, outu_hbm, outi_hbm,
                    idxu_v, idxi_v, su_v, si_v, obu_v, obi_v, usem, isem,
                    osem):
    wid = lax.axis_index("s") * NC + lax.axis_index("c")
    base = wid * E_PER_W

    def chunk_body(ch, _):
        cbase = base + ch * CHUNK
        pltpu.sync_copy(users_hbm.at[pl.ds(cbase, CHUNK)], idxu_v)
        pltpu.sync_copy(items_hbm.at[pl.ds(cbase, CHUNK)], idxi_v)
        uvecs = [idxu_v[pl.ds(k * LANES, LANES)] for k in range(CHUNK // LANES)]
        ivecs = [idxi_v[pl.ds(k * LANES, LANES)] for k in range(CHUNK // LANES)]

        def idx_of(e):
            return uvecs[e // LANES][e % LANES], ivecs[e // LANES][e % LANES]

        def start(p):
            slot = p % DEPTH
            s = slot * PAIR
            copies = []
            for j in range(PAIR):
                ru, ri = idx_of(p * PAIR + j)
                au = pl.multiple_of((ru >> 7) * 128, 128)
                ai = pl.multiple_of((ri >> 7) * 128, 128)
                copies.append(pltpu.async_copy(
                    ut_hbm.at[:, pl.ds(au, 128)], su_v.at[s + j],
                    usem.at[slot]))
                copies.append(pltpu.async_copy(
                    it_hbm.at[:, pl.ds(ai, 128)], si_v.at[s + j],
                    isem.at[slot]))
            return copies

        def finish(p, copies):
            for c in copies:
                c.wait()
            s = (p % DEPTH) * PAIR
            for j in range(PAIR):
                e = p * PAIR + j
                ru, ri = idx_of(e)
                _extract_col(su_v, s + j, ru & 127, obu_v, e)
                _extract_col(si_v, s + j, ri & 127, obi_v, e)

        n_pairs = CHUNK // PAIR
        pending = []
        for p in range(n_pairs):
            pending.append((p, start(p)))
            if len(pending) == DEPTH:
                fp, fc = pending.pop(0)
                finish(fp, fc)
        for fp, fc in pending:
            finish(fp, fc)

        ou = pltpu.async_copy(obu_v, outu_hbm.at[pl.ds(cbase, CHUNK)], osem)
        oi = pltpu.async_copy(obi_v, outi_hbm.at[pl.ds(cbase, CHUNK)], osem)
        ou.wait()
        oi.wait()
        return 0

    lax.fori_loop(0, N_CHUNKS, chunk_body, 0)


def _sc_gather(users, items, user_table_t, item_table_t):
    mesh = plsc.VectorSubcoreMesh(core_axis_name="c", subcore_axis_name="s")
    k = pl.kernel(
        _sc_gather_body,
        out_type=[
            jax.ShapeDtypeStruct((BATCH, EMBED), jnp.float32),
            jax.ShapeDtypeStruct((BATCH, EMBED), jnp.float32),
        ],
        mesh=mesh,
        compiler_params=pltpu.CompilerParams(
            use_tc_tiling_on_sc=True, needs_layout_passes=False),
        scratch_types=[
            pltpu.VMEM((CHUNK,), jnp.int32),
            pltpu.VMEM((CHUNK,), jnp.int32),
            pltpu.VMEM((DEPTH * PAIR, EMBED, 128), jnp.float32),
            pltpu.VMEM((DEPTH * PAIR, EMBED, 128), jnp.float32),
            pltpu.VMEM((CHUNK, EMBED), jnp.float32),
            pltpu.VMEM((CHUNK, EMBED), jnp.float32),
            pltpu.SemaphoreType.DMA((DEPTH,)),
            pltpu.SemaphoreType.DMA((DEPTH,)),
            pltpu.SemaphoreType.DMA,
        ],
    )
    return k(users, items, user_table_t, item_table_t)


def _tc_score_body(u_ref, i_ref, o_ref):
    prod = u_ref[...] * i_ref[...]
    s = jnp.sum(prod, axis=1, keepdims=True)
    o_ref[...] = 1.0 / (1.0 + jnp.exp(-s))


def _tc_score(u_s, i_s):
    blk = 2048
    return pl.pallas_call(
        _tc_score_body,
        grid=(BATCH // blk,),
        in_specs=[
            pl.BlockSpec((blk, EMBED), lambda i: (i, 0)),
            pl.BlockSpec((blk, EMBED), lambda i: (i, 0)),
        ],
        out_specs=pl.BlockSpec((blk, 1), lambda i: (i, 0)),
        out_shape=jax.ShapeDtypeStruct((BATCH, 1), jnp.float32),
    )(u_s, i_s)


@jax.jit
def kernel(users, items, user_table, item_table):
    u_s, i_s = _sc_gather(users, items, user_table.T, item_table.T)
    return _tc_score(u_s, i_s).reshape(BATCH)
